# baseline (speedup 1.0000x reference)
import functools

import jax
import jax.numpy as jnp
from jax import lax
from jax.experimental import pallas as pl
from jax.experimental.pallas import tpu as pltpu
from jax.experimental.pallas import tpu_sc as plsc

NC = 2
NS = 16
L = 16
NW = NC * NS

CHUNK = 128
GROUPS = CHUNK // L


def _tec_body(motifs_hbm, reward_hbm, table_hbm, out_hbm,
              sidx_v, rows0_v, rows1_v, rew_v, part_v, acc_v,
              sem_in, sem0, sem1):
    m, b = motifs_hbm.shape
    _, d = table_hbm.shape
    b_per_w = b // NW
    nchunk = b_per_w // CHUNK

    wid = lax.axis_index("s") * NC + lax.axis_index("c")
    iota = lax.iota(jnp.int32, L)

    def stage_idx(t, sem):
        return [
            pltpu.async_copy(
                motifs_hbm.at[pl.ds(mm, 1),
                              pl.ds(wid * b_per_w + t * CHUNK, CHUNK)],
                sidx_v.at[pl.ds(t * m + mm, 1)], sem)
            for mm in range(m)
        ]

    idx0_hs = stage_idx(0, sem0)
    rest_hs = [h for t in range(1, nchunk) for h in stage_idx(t, sem_in)]
    rew_h = pltpu.async_copy(reward_hbm.at[pl.ds(wid * b_per_w, b_per_w)],
                             rew_v, sem1)

    rows_bufs = (rows0_v, rows1_v)
    sems = (sem0, sem1)

    def fire(t):
        buf, sem = rows_bufs[t % 2], sems[t % 2]
        return [
            pltpu.async_copy(table_hbm.at[sidx_v.at[t * m + mm]], buf.at[mm], sem)
            for mm in range(m)
        ]

    for h in idx0_hs:
        h.wait()
    handles = fire(0)
    for h in rest_hs:
        h.wait()
    rew_h.wait()
    loss_acc = jnp.zeros((L,), jnp.float32)
    for t in range(nchunk):
        nxt = fire(t + 1) if t + 1 < nchunk else None
        for h in handles:
            h.wait()
        buf = rows_bufs[t % 2]

        @plsc.parallel_loop(0, CHUNK, unroll=4)
        def _(e, buf=buf):
            acc = jnp.zeros((L,), jnp.float32)
            for c in range(d // L):
                sl = pl.ds(c * L, L)
                prod = buf[0, e, sl]
                for mm in range(1, m):
                    prod = prod * buf[mm, e, sl]
                acc = acc + prod
            part_v[pl.ds(e * L, L)] = acc

        def group_body(g, loss_acc, t=t):
            rowbase = (g * L + iota) * L
            score = jnp.zeros((L,), jnp.float32)
            for j in range(L):
                score = score + plsc.load_gather(part_v, [rowbase + j])
            p = jnp.clip(1.0 - jnp.exp(-score), 1e-5, 1.0)
            rew = rew_v[pl.ds(t * CHUNK + g * L, L)]
            return loss_acc + p * rew
        loss_acc = plsc.parallel_loop(0, GROUPS, carry=loss_acc)(group_body)
        handles = nxt

    acc_v[...] = loss_acc
    pltpu.sync_copy(acc_v, out_hbm.at[wid])


@jax.jit
def _run_sc(motifs, reward, embedding_matrix):
    b, m = motifs.shape
    _, d = embedding_matrix.shape
    b_per_w = b // NW
    mesh = plsc.VectorSubcoreMesh(core_axis_name="c", subcore_axis_name="s")
    kern = functools.partial(
        pl.kernel,
        mesh=mesh,
        compiler_params=pltpu.CompilerParams(needs_layout_passes=False),
        out_type=jax.ShapeDtypeStruct((NW, L), jnp.float32),
        scratch_types=[
            pltpu.VMEM(((b_per_w // CHUNK) * m, CHUNK), jnp.int32),
            pltpu.VMEM((m, CHUNK, d), jnp.float32),
            pltpu.VMEM((m, CHUNK, d), jnp.float32),
            pltpu.VMEM((b_per_w,), jnp.float32),
            pltpu.VMEM((CHUNK * L,), jnp.float32),
            pltpu.VMEM((L,), jnp.float32),
            pltpu.SemaphoreType.DMA,
            pltpu.SemaphoreType.DMA,
            pltpu.SemaphoreType.DMA,
        ],
    )(_tec_body)
    return kern(motifs.T, reward, embedding_matrix)


def kernel(motifs, reward, embedding_matrix):
    partials = _run_sc(motifs, reward, embedding_matrix)
    return -(jnp.sum(partials) / motifs.shape[0])


# speedup vs baseline: 1.0223x; 1.0223x over previous
import functools

import jax
import jax.numpy as jnp
from jax import lax
from jax.experimental import pallas as pl
from jax.experimental.pallas import tpu as pltpu
from jax.experimental.pallas import tpu_sc as plsc

NC = 2
NS = 16
L = 16
NW = NC * NS

CHUNK = 128
GROUPS = CHUNK // L


def _tec_body(motifs_hbm, reward_hbm, table_hbm, out_hbm,
              sidx_v, rows0_v, rows1_v, rew_v, part_v, acc_v,
              sem_in, sem0, sem1):
    m, b = motifs_hbm.shape
    _, d = table_hbm.shape
    b_per_w = b // NW
    nchunk = b_per_w // CHUNK

    wid = lax.axis_index("s") * NC + lax.axis_index("c")
    iota = lax.iota(jnp.int32, L)

    def stage_idx(t, sem):
        return [
            pltpu.async_copy(
                motifs_hbm.at[pl.ds(mm, 1),
                              pl.ds(wid * b_per_w + t * CHUNK, CHUNK)],
                sidx_v.at[pl.ds(t * m + mm, 1)], sem)
            for mm in range(m)
        ]

    idx0_hs = stage_idx(0, sem0)
    rest_hs = [h for t in range(1, nchunk) for h in stage_idx(t, sem_in)]
    rew_h = pltpu.async_copy(reward_hbm.at[pl.ds(wid * b_per_w, b_per_w)],
                             rew_v, sem1)

    rows_bufs = (rows0_v, rows1_v)
    sems = (sem0, sem1)

    def fire(t):
        buf, sem = rows_bufs[t % 2], sems[t % 2]
        return [
            pltpu.async_copy(table_hbm.at[sidx_v.at[t * m + mm]], buf.at[mm], sem)
            for mm in range(m)
        ]

    for h in idx0_hs:
        h.wait()
    handles = fire(0)
    for h in rest_hs:
        h.wait()
    rew_h.wait()
    loss_acc = jnp.zeros((L,), jnp.float32)
    for t in range(nchunk):
        nxt = fire(t + 1) if t + 1 < nchunk else None
        for h in handles:
            h.wait()
        buf = rows_bufs[t % 2]

        @plsc.parallel_loop(0, CHUNK, unroll=2)
        def _(e, buf=buf):
            acc = jnp.zeros((L,), jnp.float32)
            for c in range(d // L):
                sl = pl.ds(c * L, L)
                prod = buf[0, e, sl]
                for mm in range(1, m):
                    prod = prod * buf[mm, e, sl]
                acc = acc + prod
            part_v[pl.ds(e * L, L)] = acc

        def group_body(g, loss_acc, t=t):
            rowbase = (g * L + iota) * L
            score = jnp.zeros((L,), jnp.float32)
            for j in range(L):
                score = score + plsc.load_gather(part_v, [rowbase + j])
            p = jnp.clip(1.0 - jnp.exp(-score), 1e-5, 1.0)
            rew = rew_v[pl.ds(t * CHUNK + g * L, L)]
            return loss_acc + p * rew
        loss_acc = plsc.parallel_loop(0, GROUPS, carry=loss_acc)(group_body)
        handles = nxt

    acc_v[...] = loss_acc
    pltpu.sync_copy(acc_v, out_hbm.at[wid])


@jax.jit
def _run_sc(motifs, reward, embedding_matrix):
    b, m = motifs.shape
    _, d = embedding_matrix.shape
    b_per_w = b // NW
    mesh = plsc.VectorSubcoreMesh(core_axis_name="c", subcore_axis_name="s")
    kern = functools.partial(
        pl.kernel,
        mesh=mesh,
        compiler_params=pltpu.CompilerParams(needs_layout_passes=False),
        out_type=jax.ShapeDtypeStruct((NW, L), jnp.float32),
        scratch_types=[
            pltpu.VMEM(((b_per_w // CHUNK) * m, CHUNK), jnp.int32),
            pltpu.VMEM((m, CHUNK, d), jnp.float32),
            pltpu.VMEM((m, CHUNK, d), jnp.float32),
            pltpu.VMEM((b_per_w,), jnp.float32),
            pltpu.VMEM((CHUNK * L,), jnp.float32),
            pltpu.VMEM((L,), jnp.float32),
            pltpu.SemaphoreType.DMA,
            pltpu.SemaphoreType.DMA,
            pltpu.SemaphoreType.DMA,
        ],
    )(_tec_body)
    return kern(motifs.T, reward, embedding_matrix)


def kernel(motifs, reward, embedding_matrix):
    partials = _run_sc(motifs, reward, embedding_matrix)
    return -(jnp.sum(partials) / motifs.shape[0])
